# baseline (speedup 1.0000x reference)
import functools

import jax
import jax.numpy as jnp
from jax import lax
from jax.experimental import pallas as pl
from jax.experimental.pallas import tpu as pltpu
from jax.experimental.pallas import tpu_sc as plsc

NUM_HEADS = 16
VOCAB = 65536
HF = 2
BATCH = 16384
MLP_DIM = 64
OUT_DIM = 3
IN_FEATS = NUM_HEADS * HF

NC = 2
NS = 16
NW = NC * NS
LANES = 16

HALF_V = VOCAB // 2
HALF_W = HALF_V * HF
PAD = LANES


def _sc_gather_body(table_hbm, idx_hbm, out_hbm, tbl_v, idx_v, o0_v, o1_v,
                    sem):
  wid = lax.axis_index("s") * NC + lax.axis_index("c")
  head = wid // 2
  half = wid % 2

  c1 = pltpu.async_copy(
      table_hbm.at[pl.ds(head * (VOCAB * HF) + half * HALF_W, HALF_W)],
      tbl_v.at[pl.ds(0, HALF_W)], sem)
  c2 = pltpu.async_copy(idx_hbm.at[head], idx_v, sem)
  tbl_v[pl.ds(HALF_W, LANES)] = jnp.zeros((LANES,), jnp.float32)
  c1.wait()
  c2.wait()

  base = half * HALF_V

  @pl.loop(0, BATCH // LANES)
  def step(i):
    sl = pl.ds(i * LANES, LANES)
    loc = idx_v[sl] - base
    m = loc.astype(jnp.uint32) < jnp.uint32(HALF_V)
    g0 = jnp.where(m, loc, HALF_V) * 2
    o0_v[sl] = plsc.load_gather(tbl_v, [g0])
    o1_v[sl] = plsc.load_gather(tbl_v, [g0 + 1])

  r0 = half * IN_FEATS + 2 * head
  pltpu.sync_copy(o0_v, out_hbm.at[r0])
  pltpu.sync_copy(o1_v, out_hbm.at[r0 + 1])


_sc_gather = functools.partial(
    pl.kernel,
    out_type=jax.ShapeDtypeStruct((2 * IN_FEATS, BATCH), jnp.float32),
    mesh=plsc.VectorSubcoreMesh(core_axis_name="c", subcore_axis_name="s"),
    scratch_types=[
        pltpu.VMEM((HALF_W + PAD,), jnp.float32),
        pltpu.VMEM((BATCH,), jnp.int32),
        pltpu.VMEM((BATCH,), jnp.float32),
        pltpu.VMEM((BATCH,), jnp.float32),
        pltpu.SemaphoreType.DMA,
    ],
    compiler_params=pltpu.CompilerParams(
        use_tc_tiling_on_sc=False, needs_layout_passes=False),
)(_sc_gather_body)


def _mlp_body(x_ref, w1t, b1, w2t, b2, w3t, b3, w4t, b4, o_ref):
  x = x_ref[...]
  x = jnp.maximum(
      jnp.dot(w1t[...], x, preferred_element_type=jnp.float32) + b1[...], 0.0)
  x = jnp.maximum(
      jnp.dot(w2t[...], x, preferred_element_type=jnp.float32) + b2[...], 0.0)
  x = jnp.maximum(
      jnp.dot(w3t[...], x, preferred_element_type=jnp.float32) + b3[...], 0.0)
  o_ref[...] = (
      jnp.dot(w4t[...], x, preferred_element_type=jnp.float32) + b4[...])


BBLK = 16384


def _mlp_t(emb, W1t, b1, W2t, b2, W3t, b3, W4t, b4):
  full = lambda i: (0, 0)
  return pl.pallas_call(
      _mlp_body,
      grid=(BATCH // BBLK,),
      in_specs=[
          pl.BlockSpec((2 * IN_FEATS, BBLK), lambda i: (0, i)),
          pl.BlockSpec((MLP_DIM, 2 * IN_FEATS), full),
          pl.BlockSpec((MLP_DIM, 1), full),
          pl.BlockSpec((MLP_DIM, MLP_DIM), full),
          pl.BlockSpec((MLP_DIM, 1), full),
          pl.BlockSpec((MLP_DIM, MLP_DIM), full),
          pl.BlockSpec((MLP_DIM, 1), full),
          pl.BlockSpec((OUT_DIM, MLP_DIM), full),
          pl.BlockSpec((OUT_DIM, 1), full),
      ],
      out_specs=pl.BlockSpec((OUT_DIM, BBLK), lambda i: (0, i)),
      out_shape=jax.ShapeDtypeStruct((OUT_DIM, BATCH), jnp.float32),
  )(emb, W1t, b1, W2t, b2, W3t, b3, W4t, b4)


@jax.jit
def kernel(input, tables, W1, b1, W2, b2, W3, b3, W4, b4):
  flat_tables = tables.reshape(NUM_HEADS * VOCAB * HF)
  idx_t = input.T
  emb = _sc_gather(flat_tables, idx_t)
  W1t2 = jnp.concatenate([W1.T, W1.T], axis=1)
  out_t = _mlp_t(emb, W1t2, b1.reshape(MLP_DIM, 1), W2.T,
                 b2.reshape(MLP_DIM, 1), W3.T, b3.reshape(MLP_DIM, 1),
                 W4.T, b4.reshape(OUT_DIM, 1))
  return out_t.T


# speedup vs baseline: 29.1918x; 29.1918x over previous
import functools

import jax
import jax.numpy as jnp
from jax import lax
from jax.experimental import pallas as pl
from jax.experimental.pallas import tpu as pltpu
from jax.experimental.pallas import tpu_sc as plsc

NUM_HEADS = 16
VOCAB = 65536
HF = 2
BATCH = 16384
MLP_DIM = 64
OUT_DIM = 3
IN_FEATS = NUM_HEADS * HF

NC = 2
NS = 16
NW = NC * NS
LANES = 16


def _sc_gather_body(table_hbm, idx_hbm, out_hbm, tbl_v, idx_v, out_v, sem):
  wid = lax.axis_index("s") * NC + lax.axis_index("c")
  head = wid // 2

  c1 = pltpu.async_copy(table_hbm.at[wid], tbl_v, sem)
  c2 = pltpu.async_copy(idx_hbm.at[head], idx_v, sem)
  c1.wait()
  c2.wait()

  HALF = BATCH // 2

  @pl.loop(0, HALF // LANES)
  def step(i):
    sl0 = pl.ds(i * LANES, LANES)
    sl1 = pl.ds(HALF + i * LANES, LANES)
    out_v[sl0] = plsc.load_gather(tbl_v, [idx_v[sl0]])
    out_v[sl1] = plsc.load_gather(tbl_v, [idx_v[sl1]])

  pltpu.sync_copy(out_v, out_hbm.at[wid])


_sc_gather = functools.partial(
    pl.kernel,
    out_type=jax.ShapeDtypeStruct((NW, BATCH), jnp.float32),
    mesh=plsc.VectorSubcoreMesh(core_axis_name="c", subcore_axis_name="s"),
    scratch_types=[
        pltpu.VMEM((VOCAB,), jnp.float32),
        pltpu.VMEM((BATCH,), jnp.int32),
        pltpu.VMEM((BATCH,), jnp.float32),
        pltpu.SemaphoreType.DMA,
    ],
    compiler_params=pltpu.CompilerParams(
        use_tc_tiling_on_sc=False, needs_layout_passes=False),
)(_sc_gather_body)


def _mlp_body(x_ref, w1t, b1, w2t, b2, w3t, b3, w4t, b4, o_ref):
  x = x_ref[...]
  x = jnp.maximum(
      jnp.dot(w1t[...], x, preferred_element_type=jnp.float32) + b1[...], 0.0)
  x = jnp.maximum(
      jnp.dot(w2t[...], x, preferred_element_type=jnp.float32) + b2[...], 0.0)
  x = jnp.maximum(
      jnp.dot(w3t[...], x, preferred_element_type=jnp.float32) + b3[...], 0.0)
  o_ref[...] = (
      jnp.dot(w4t[...], x, preferred_element_type=jnp.float32) + b4[...])


BBLK = 16384


def _mlp_t(emb, W1t, b1, W2t, b2, W3t, b3, W4t, b4):
  full = lambda i: (0, 0)
  return pl.pallas_call(
      _mlp_body,
      grid=(BATCH // BBLK,),
      in_specs=[
          pl.BlockSpec((IN_FEATS, BBLK), lambda i: (0, i)),
          pl.BlockSpec((MLP_DIM, IN_FEATS), full),
          pl.BlockSpec((MLP_DIM, 1), full),
          pl.BlockSpec((MLP_DIM, MLP_DIM), full),
          pl.BlockSpec((MLP_DIM, 1), full),
          pl.BlockSpec((MLP_DIM, MLP_DIM), full),
          pl.BlockSpec((MLP_DIM, 1), full),
          pl.BlockSpec((OUT_DIM, MLP_DIM), full),
          pl.BlockSpec((OUT_DIM, 1), full),
      ],
      out_specs=pl.BlockSpec((OUT_DIM, BBLK), lambda i: (0, i)),
      out_shape=jax.ShapeDtypeStruct((OUT_DIM, BATCH), jnp.float32),
  )(emb, W1t, b1, W2t, b2, W3t, b3, W4t, b4)


@jax.jit
def kernel(input, tables, W1, b1, W2, b2, W3, b3, W4, b4):
  tables_t = tables.transpose(0, 2, 1).reshape(NW, VOCAB)
  idx_t = input.T
  emb = _sc_gather(tables_t, idx_t)
  out_t = _mlp_t(emb, W1.T, b1.reshape(MLP_DIM, 1), W2.T,
                 b2.reshape(MLP_DIM, 1), W3.T, b3.reshape(MLP_DIM, 1),
                 W4.T, b4.reshape(OUT_DIM, 1))
  return out_t.T
